# baseline (speedup 1.0000x reference)
import functools

import jax
import jax.numpy as jnp
from jax import lax
from jax.experimental import pallas as pl
from jax.experimental.pallas import tpu as pltpu
from jax.experimental.pallas import tpu_sc as plsc

N_NODES = 50000
N_EDGES = 800000
HD = 64
OUT_DIM = 128
N_LAYERS = 3

NT = 32
NSUB = 16
BLK = 256
NP = 50176
GRID = NP // BLK
HALF = NP // 2
SH = HALF + 512
ZPT = SH // NSUB
OPT = HALF // NSUB
OCH = 56
K = 64
CPT = 782
EP = NSUB * CPT * K
GK = 112
GCH = (NP // NT) // GK
BIG = 1 << 28

_mesh = plsc.VectorSubcoreMesh(core_axis_name="c", subcore_axis_name="s")


@functools.partial(
    pl.kernel,
    out_type=jax.ShapeDtypeStruct((NP, 2 * HD), jnp.float32),
    mesh=_mesh,
    compiler_params=pltpu.CompilerParams(use_tc_tiling_on_sc=False),
    scratch_types=[
        pltpu.VMEM((GK,), jnp.int32),
        pltpu.VMEM((GK, 2 * HD), jnp.float32),
        pltpu.SemaphoreType.DMA,
    ],
)
def _sc_embed(an_hbm, emb_hbm, x_hbm, idx_v, rows_v, sem):
    wid = lax.axis_index("s") * 2 + lax.axis_index("c")

    def body(t, carry):
        base = wid * (NP // NT) + t * GK
        pltpu.sync_copy(an_hbm.at[pl.ds(base, GK)], idx_v)
        pltpu.async_copy(emb_hbm.at[idx_v], rows_v, sem).wait()
        pltpu.sync_copy(rows_v, x_hbm.at[pl.ds(base, GK)])
        return carry

    lax.fori_loop(0, GCH, body, 0)


@functools.partial(
    pl.kernel,
    out_type=jax.ShapeDtypeStruct((NP, HD), jnp.float32),
    mesh=_mesh,
    compiler_params=pltpu.CompilerParams(use_tc_tiling_on_sc=False),
    scratch_types=[
        pltpu.VMEM((K,), jnp.int32),
        pltpu.VMEM((K,), jnp.int32),
        pltpu.VMEM((K,), jnp.int32),
        pltpu.VMEM((K,), jnp.int32),
        pltpu.VMEM((K, 2 * HD), jnp.float32),
        pltpu.VMEM((K, 2 * HD), jnp.float32),
        pltpu.VMEM((K, HD), jnp.float32),
        pltpu.VMEM_SHARED((SH, HD), jnp.float32),
        pltpu.SemaphoreType.DMA,
        pltpu.SemaphoreType.DMA,
    ],
)
def _sc_edge(a_hbm, bc_hbm, row_hbm, col_hbm, s_hbm,
             irow, icol, ig, il, abuf, bcbuf, mbuf, acc, sem_a, sem_bc):
    c = lax.axis_index("c")
    s = lax.axis_index("s")
    base = c * HALF

    def zero_mbuf(e, carry):
        for j in range(HD // 16):
            mbuf[e, pl.ds(j * 16, 16)] = jnp.zeros((16,), jnp.float32)
        return carry

    lax.fori_loop(0, K, zero_mbuf, 0)

    def zero_acc(t, carry):
        row0 = s * ZPT + t * K

        def mkrows(i, carry2):
            il[pl.ds(i * 16, 16)] = row0 + i * 16 + lax.iota(jnp.int32, 16)
            return carry2

        lax.fori_loop(0, K // 16, mkrows, 0)
        pltpu.sync_copy(mbuf, acc.at[il])
        return carry

    lax.fori_loop(0, ZPT // K, zero_acc, 0)
    plsc.subcore_barrier()

    def chunk(t, carry):
        e0 = (s * CPT + t) * K
        pltpu.sync_copy(row_hbm.at[pl.ds(e0, K)], irow)
        pltpu.sync_copy(col_hbm.at[pl.ds(e0, K)], icol)

        def mkidx(i, carry2):
            off = pl.ds(i * 16, 16)
            r = irow[off]
            ig[off] = jnp.where(r < N_NODES, r, 0)
            l = r - base
            ok = (l >= 0) & (l < HALF)
            il[off] = jnp.where(ok, l, HALF)
            return carry2

        lax.fori_loop(0, K // 16, mkidx, 0)

        cp_a = pltpu.async_copy(a_hbm.at[ig], abuf, sem_a)
        cp_bc = pltpu.async_copy(bc_hbm.at[icol], bcbuf, sem_bc)
        cp_a.wait()
        cp_bc.wait()

        def msg(e, carry2):
            for j in range(HD // 16):
                jo = pl.ds(j * 16, 16)
                av = abuf[e, jo]
                bv = bcbuf[e, jo]
                cv = bcbuf[e, pl.ds(HD + j * 16, 16)]
                g = 1.0 / (1.0 + jnp.exp(-(av + bv)))
                mbuf[e, jo] = g * cv
            return carry2

        lax.fori_loop(0, K, msg, 0)
        pltpu.sync_copy(mbuf, acc.at[il], add=True)
        return carry

    lax.fori_loop(0, CPT, chunk, 0)
    plsc.subcore_barrier()

    def drain(t, carry):
        row0 = s * OPT + t * OCH

        def mkrows(i, carry2):
            pos = i * 16 + lax.iota(jnp.int32, 16)
            il[pl.ds(i * 16, 16)] = jnp.where(pos < OCH, row0 + pos, HALF)
            return carry2

        lax.fori_loop(0, K // 16, mkrows, 0)
        pltpu.async_copy(acc.at[il], mbuf, sem_a).wait()
        pltpu.sync_copy(mbuf.at[pl.ds(0, OCH)],
                        s_hbm.at[pl.ds(base + row0, OCH)])
        return carry

    lax.fori_loop(0, OPT // OCH, drain, 0)


def _dense_first_body(x_ref, wgr_ref, wgc_ref, wl_ref, bg_ref, bl_ref,
                      a_ref, bc_ref):
    x = x_ref[:, :HD]
    a_ref[:, :HD] = jnp.dot(x, wgr_ref[...], preferred_element_type=jnp.float32)
    a_ref[:, HD:] = jnp.zeros((BLK, HD), jnp.float32)
    bc_ref[:, :HD] = (jnp.dot(x, wgc_ref[...],
                              preferred_element_type=jnp.float32) + bg_ref[...])
    bc_ref[:, HD:] = (jnp.dot(x, wl_ref[...],
                              preferred_element_type=jnp.float32) + bl_ref[...])


def _dense_mid_body(s_ref, bcp_ref, wgr_ref, wgc_ref, wl_ref, bg_ref, bl_ref,
                    a_ref, bc_ref):
    x = jnp.maximum(s_ref[...] + bcp_ref[:, HD:], 0.0)
    a_ref[:, :HD] = jnp.dot(x, wgr_ref[...], preferred_element_type=jnp.float32)
    a_ref[:, HD:] = jnp.zeros((BLK, HD), jnp.float32)
    bc_ref[:, :HD] = (jnp.dot(x, wgc_ref[...],
                              preferred_element_type=jnp.float32) + bg_ref[...])
    bc_ref[:, HD:] = (jnp.dot(x, wl_ref[...],
                              preferred_element_type=jnp.float32) + bl_ref[...])


_row_spec = pl.BlockSpec((BLK, HD), lambda i: (i, 0))
_bc_spec = pl.BlockSpec((BLK, 2 * HD), lambda i: (i, 0))
_w_spec = pl.BlockSpec((HD, HD), lambda i: (0, 0))
_b_spec = pl.BlockSpec((1, HD), lambda i: (0, 0))

_dense_out = (jax.ShapeDtypeStruct((NP, 2 * HD), jnp.float32),
              jax.ShapeDtypeStruct((NP, 2 * HD), jnp.float32))
_dense_out_specs = [_bc_spec, _bc_spec]

_tc_dense_first = pl.pallas_call(
    _dense_first_body,
    grid=(GRID,),
    in_specs=[_bc_spec, _w_spec, _w_spec, _w_spec, _b_spec, _b_spec],
    out_specs=_dense_out_specs,
    out_shape=_dense_out,
)

_tc_dense_mid = pl.pallas_call(
    _dense_mid_body,
    grid=(GRID,),
    in_specs=[_row_spec, _bc_spec, _w_spec, _w_spec, _w_spec, _b_spec, _b_spec],
    out_specs=_dense_out_specs,
    out_shape=_dense_out,
)


def _pool_body(s_ref, bcp_ref, w1_ref, b1_ref, w2_ref, b2_ref, o_ref, acc_ref):
    pid = pl.program_id(0)
    x = jnp.maximum(s_ref[...] + bcp_ref[:, HD:], 0.0)
    gid = pid * BLK + lax.broadcasted_iota(jnp.int32, (BLK, 1), 0)
    x = jnp.where(gid < N_NODES, x, 0.0)

    @pl.when(pid == 0)
    def _():
        acc_ref[...] = x

    @pl.when(pid != 0)
    def _():
        acc_ref[...] = acc_ref[...] + x

    @pl.when(pid == GRID - 1)
    def _():
        h = jnp.sum(acc_ref[...], axis=0, keepdims=True) * (1.0 / N_NODES)
        h1 = jnp.maximum(
            jnp.dot(h, w1_ref[...], preferred_element_type=jnp.float32)
            + b1_ref[...], 0.0)
        o_ref[...] = (jnp.dot(h1, w2_ref[...],
                              preferred_element_type=jnp.float32) + b2_ref[...])


_tc_pool = pl.pallas_call(
    _pool_body,
    grid=(GRID,),
    in_specs=[_row_spec, _bc_spec, _w_spec,
              pl.BlockSpec((1, HD), lambda i: (0, 0)),
              pl.BlockSpec((HD, OUT_DIM), lambda i: (0, 0)),
              pl.BlockSpec((1, OUT_DIM), lambda i: (0, 0))],
    out_specs=pl.BlockSpec((1, OUT_DIM), lambda i: (0, 0)),
    out_shape=jax.ShapeDtypeStruct((1, OUT_DIM), jnp.float32),
    scratch_shapes=[pltpu.VMEM((BLK, HD), jnp.float32)],
)


def kernel(atomic_numbers, positions, lattice, edge_index, emb, Wl, bl,
           Wg, bg, W1, b1, W2, b2):
    an = atomic_numbers.astype(jnp.int32)
    anp = jnp.concatenate([an, jnp.zeros((NP - N_NODES,), jnp.int32)])
    row = edge_index[0].astype(jnp.int32)
    col = edge_index[1].astype(jnp.int32)
    rowp = jnp.concatenate([row, jnp.full((EP - N_EDGES,), BIG, jnp.int32)])
    colp = jnp.concatenate([col, jnp.zeros((EP - N_EDGES,), jnp.int32)])
    embp = jnp.pad(emb, ((0, 4), (0, HD)))

    x0 = _sc_embed(anp, embp)
    A, BC = _tc_dense_first(x0, Wg[0, :HD], Wg[0, HD:], Wl[0],
                            bg[0].reshape(1, HD), bl[0].reshape(1, HD))
    S = _sc_edge(A, BC, rowp, colp)
    for i in range(1, N_LAYERS):
        A, BC = _tc_dense_mid(S, BC, Wg[i, :HD], Wg[i, HD:], Wl[i],
                              bg[i].reshape(1, HD), bl[i].reshape(1, HD))
        S = _sc_edge(A, BC, rowp, colp)
    out = _tc_pool(S, BC, W1, b1.reshape(1, HD), W2, b2.reshape(1, OUT_DIM))
    return out.reshape(OUT_DIM)


# speedup vs baseline: 1.3532x; 1.3532x over previous
import functools

import jax
import jax.numpy as jnp
from jax import lax
from jax.experimental import pallas as pl
from jax.experimental.pallas import tpu as pltpu
from jax.experimental.pallas import tpu_sc as plsc

N_NODES = 50000
N_EDGES = 800000
HD = 64
OUT_DIM = 128
N_LAYERS = 3

NT = 32
NSUB = 16
BLK = 256
NP = 50176
GRID = NP // BLK
HALF = NP // 2
SH = HALF + 64
ZPT = SH // NSUB
OPT = HALF // NSUB
OCH = 56
K = 64
G = 16
CPT = 784
EP = NSUB * CPT * K + G * K
GK = 112
GCH = (NP // NT) // GK
BIG = 1 << 28

_mesh = plsc.VectorSubcoreMesh(core_axis_name="c", subcore_axis_name="s")


@functools.partial(
    pl.kernel,
    out_type=jax.ShapeDtypeStruct((NP, HD), jnp.float32),
    mesh=_mesh,
    compiler_params=pltpu.CompilerParams(use_tc_tiling_on_sc=False),
    scratch_types=[
        pltpu.VMEM((GK,), jnp.int32),
        pltpu.VMEM((GK, HD), jnp.float32),
        pltpu.SemaphoreType.DMA,
    ],
)
def _sc_embed(an_hbm, emb_hbm, x_hbm, idx_v, rows_v, sem):
    wid = lax.axis_index("s") * 2 + lax.axis_index("c")

    def body(t, carry):
        base = wid * (NP // NT) + t * GK
        pltpu.sync_copy(an_hbm.at[pl.ds(base, GK)], idx_v)
        pltpu.async_copy(emb_hbm.at[idx_v], rows_v, sem).wait()
        pltpu.sync_copy(rows_v, x_hbm.at[pl.ds(base, GK)])
        return carry

    lax.fori_loop(0, GCH, body, 0)


@functools.partial(
    pl.kernel,
    out_type=jax.ShapeDtypeStruct((NP, HD), jnp.float32),
    mesh=_mesh,
    compiler_params=pltpu.CompilerParams(use_tc_tiling_on_sc=False),
    scratch_types=[
        pltpu.VMEM((G * K,), jnp.int32),
        pltpu.VMEM((G * K,), jnp.int32),
        [pltpu.VMEM((K,), jnp.int32)] * 2,
        [pltpu.VMEM((K,), jnp.int32)] * 2,
        [pltpu.VMEM((K,), jnp.int32)] * 2,
        [pltpu.VMEM((K, HD), jnp.float32)] * 2,
        [pltpu.VMEM((K, 2 * HD), jnp.float32)] * 2,
        pltpu.VMEM_SHARED((SH, HD), jnp.float32),
        [pltpu.SemaphoreType.DMA] * 2,
        [pltpu.SemaphoreType.DMA] * 2,
    ],
)
def _sc_edge(a_hbm, bc_hbm, row_hbm, col_hbm, s_hbm,
             irow_big, icol_big, ig, ick, il, ab, bc, acc, sema, semb):
    c = lax.axis_index("c")
    s = lax.axis_index("s")
    base = c * HALF
    eb = s * CPT * K

    def zero_ab(e, carry):
        for j in range(HD // 16):
            ab[0][e, pl.ds(j * 16, 16)] = jnp.zeros((16,), jnp.float32)
        return carry

    lax.fori_loop(0, K, zero_ab, 0)

    def zero_acc(t, carry):
        def mkrows(i, carry2):
            pos = t * K + i * 16 + lax.iota(jnp.int32, 16)
            il[0][pl.ds(i * 16, 16)] = jnp.where(pos < ZPT, s * ZPT + pos, HALF)
            return carry2

        lax.fori_loop(0, K // 16, mkrows, 0)
        pltpu.sync_copy(ab[0], acc.at[il[0]])
        return carry

    lax.fori_loop(0, (ZPT + K - 1) // K, zero_acc, 0)
    plsc.subcore_barrier()

    def load_group(tn):
        off = eb + tn * K
        pltpu.sync_copy(row_hbm.at[pl.ds(off, G * K)], irow_big)
        pltpu.sync_copy(col_hbm.at[pl.ds(off, G * K)], icol_big)

    def mkidx(tn, p):
        off_big = lax.rem(tn, G) * K

        def body(i, carry):
            off = pl.ds(i * 16, 16)
            r = irow_big[pl.ds(off_big + i * 16, 16)]
            cv = icol_big[pl.ds(off_big + i * 16, 16)]
            ig[p][off] = jnp.where(r < N_NODES, r, 0)
            ick[p][off] = cv
            l = r - base
            ok = (l >= 0) & (l < HALF)
            il[p][off] = jnp.where(ok, l, HALF)
            return carry

        lax.fori_loop(0, K // 16, body, 0)

    def fire(p):
        pltpu.async_copy(a_hbm.at[ig[p]], ab[p], sema[p])
        pltpu.async_copy(bc_hbm.at[ick[p]], bc[p], semb[p])

    def wait(p):
        pltpu.make_async_copy(a_hbm.at[ig[p]], ab[p], sema[p]).wait()
        pltpu.make_async_copy(bc_hbm.at[ick[p]], bc[p], semb[p]).wait()

    load_group(0)
    mkidx(0, 0)
    fire(0)

    def pair(tp, carry):
        for b in (0, 1):
            t = tp * 2 + b
            tn = t + 1
            wait(b)

            @pl.when(lax.rem(tn, G) == 0)
            def _():
                load_group(tn)

            mkidx(tn, 1 - b)
            fire(1 - b)

            def msg(e, carry2):
                for j in range(HD // 16):
                    jo = pl.ds(j * 16, 16)
                    av = ab[b][e, jo]
                    bv = bc[b][e, jo]
                    cv = bc[b][e, pl.ds(HD + j * 16, 16)]
                    g = 1.0 / (1.0 + jnp.exp(-(av + bv)))
                    ab[b][e, jo] = g * cv
                return carry2

            lax.fori_loop(0, K, msg, 0)
            pltpu.sync_copy(ab[b], acc.at[il[b]], add=True)
        return carry

    lax.fori_loop(0, CPT // 2, pair, 0)
    wait(0)
    plsc.subcore_barrier()

    def drain(t, carry):
        row0 = s * OPT + t * OCH

        def mkrows(i, carry2):
            pos = i * 16 + lax.iota(jnp.int32, 16)
            il[0][pl.ds(i * 16, 16)] = jnp.where(pos < OCH, row0 + pos, HALF)
            return carry2

        lax.fori_loop(0, K // 16, mkrows, 0)
        pltpu.async_copy(acc.at[il[0]], ab[0], sema[0]).wait()
        pltpu.sync_copy(ab[0].at[pl.ds(0, OCH)],
                        s_hbm.at[pl.ds(base + row0, OCH)])
        return carry

    lax.fori_loop(0, OPT // OCH, drain, 0)


def _dense_first_body(x_ref, wgr_ref, wgc_ref, wl_ref, bg_ref, bl_ref,
                      a_ref, bc_ref):
    x = x_ref[...]
    a_ref[...] = jnp.dot(x, wgr_ref[...], preferred_element_type=jnp.float32)
    bc_ref[:, :HD] = (jnp.dot(x, wgc_ref[...],
                              preferred_element_type=jnp.float32) + bg_ref[...])
    bc_ref[:, HD:] = (jnp.dot(x, wl_ref[...],
                              preferred_element_type=jnp.float32) + bl_ref[...])


def _dense_mid_body(s_ref, bcp_ref, wgr_ref, wgc_ref, wl_ref, bg_ref, bl_ref,
                    a_ref, bc_ref):
    x = jnp.maximum(s_ref[...] + bcp_ref[:, HD:], 0.0)
    a_ref[...] = jnp.dot(x, wgr_ref[...], preferred_element_type=jnp.float32)
    bc_ref[:, :HD] = (jnp.dot(x, wgc_ref[...],
                              preferred_element_type=jnp.float32) + bg_ref[...])
    bc_ref[:, HD:] = (jnp.dot(x, wl_ref[...],
                              preferred_element_type=jnp.float32) + bl_ref[...])


_row_spec = pl.BlockSpec((BLK, HD), lambda i: (i, 0))
_bc_spec = pl.BlockSpec((BLK, 2 * HD), lambda i: (i, 0))
_w_spec = pl.BlockSpec((HD, HD), lambda i: (0, 0))
_b_spec = pl.BlockSpec((1, HD), lambda i: (0, 0))

_dense_out = (jax.ShapeDtypeStruct((NP, HD), jnp.float32),
              jax.ShapeDtypeStruct((NP, 2 * HD), jnp.float32))
_dense_out_specs = [_row_spec, _bc_spec]

_tc_dense_first = pl.pallas_call(
    _dense_first_body,
    grid=(GRID,),
    in_specs=[_row_spec, _w_spec, _w_spec, _w_spec, _b_spec, _b_spec],
    out_specs=_dense_out_specs,
    out_shape=_dense_out,
)

_tc_dense_mid = pl.pallas_call(
    _dense_mid_body,
    grid=(GRID,),
    in_specs=[_row_spec, _bc_spec, _w_spec, _w_spec, _w_spec, _b_spec, _b_spec],
    out_specs=_dense_out_specs,
    out_shape=_dense_out,
)


def _pool_body(s_ref, bcp_ref, w1_ref, b1_ref, w2_ref, b2_ref, o_ref, acc_ref):
    pid = pl.program_id(0)
    x = jnp.maximum(s_ref[...] + bcp_ref[:, HD:], 0.0)
    gid = pid * BLK + lax.broadcasted_iota(jnp.int32, (BLK, 1), 0)
    x = jnp.where(gid < N_NODES, x, 0.0)

    @pl.when(pid == 0)
    def _():
        acc_ref[...] = x

    @pl.when(pid != 0)
    def _():
        acc_ref[...] = acc_ref[...] + x

    @pl.when(pid == GRID - 1)
    def _():
        h = jnp.sum(acc_ref[...], axis=0, keepdims=True) * (1.0 / N_NODES)
        h1 = jnp.maximum(
            jnp.dot(h, w1_ref[...], preferred_element_type=jnp.float32)
            + b1_ref[...], 0.0)
        o_ref[...] = (jnp.dot(h1, w2_ref[...],
                              preferred_element_type=jnp.float32) + b2_ref[...])


_tc_pool = pl.pallas_call(
    _pool_body,
    grid=(GRID,),
    in_specs=[_row_spec, _bc_spec, _w_spec,
              pl.BlockSpec((1, HD), lambda i: (0, 0)),
              pl.BlockSpec((HD, OUT_DIM), lambda i: (0, 0)),
              pl.BlockSpec((1, OUT_DIM), lambda i: (0, 0))],
    out_specs=pl.BlockSpec((1, OUT_DIM), lambda i: (0, 0)),
    out_shape=jax.ShapeDtypeStruct((1, OUT_DIM), jnp.float32),
    scratch_shapes=[pltpu.VMEM((BLK, HD), jnp.float32)],
)


def kernel(atomic_numbers, positions, lattice, edge_index, emb, Wl, bl,
           Wg, bg, W1, b1, W2, b2):
    an = atomic_numbers.astype(jnp.int32)
    anp = jnp.concatenate([an, jnp.zeros((NP - N_NODES,), jnp.int32)])
    row = edge_index[0].astype(jnp.int32)
    col = edge_index[1].astype(jnp.int32)
    rowp = jnp.concatenate([row, jnp.full((EP - N_EDGES,), BIG, jnp.int32)])
    colp = jnp.concatenate([col, jnp.zeros((EP - N_EDGES,), jnp.int32)])
    x0 = _sc_embed(anp, emb)
    A, BC = _tc_dense_first(x0, Wg[0, :HD], Wg[0, HD:], Wl[0],
                            bg[0].reshape(1, HD), bl[0].reshape(1, HD))
    S = _sc_edge(A, BC, rowp, colp)
    for i in range(1, N_LAYERS):
        A, BC = _tc_dense_mid(S, BC, Wg[i, :HD], Wg[i, HD:], Wl[i],
                              bg[i].reshape(1, HD), bl[i].reshape(1, HD))
        S = _sc_edge(A, BC, rowp, colp)
    out = _tc_pool(S, BC, W1, b1.reshape(1, HD), W2, b2.reshape(1, OUT_DIM))
    return out.reshape(OUT_DIM)
